# baseline (speedup 1.0000x reference)
import functools

import jax
import jax.numpy as jnp
from jax import lax
from jax.experimental import pallas as pl
from jax.experimental.pallas import tpu as pltpu
from jax.experimental.pallas import tpu_sc as plsc

D_MODEL = 768
MAX_LEN = 8192
B = 4
L = 2048

NC = 2
NS = 16
NW = NC * NS
N_GATHER = B * L
ROWS_PER_TILE = N_GATHER // NW
CHUNK = 16
LCHUNK = CHUNK // B
NCHUNK = ROWS_PER_TILE // CHUNK
LANES = 16
VECS_PER_ROW = D_MODEL // LANES
NBUF = 4
PRE = 2

_MESH = plsc.VectorSubcoreMesh(core_axis_name="c", subcore_axis_name="s")


@functools.partial(
    pl.kernel,
    out_type=jax.ShapeDtypeStruct((L + 1, B, D_MODEL), jnp.float32),
    mesh=_MESH,
    scratch_types=[
        pltpu.VMEM((NCHUNK, CHUNK), jnp.int32),
        [pltpu.VMEM((LCHUNK, B, D_MODEL), jnp.float32) for _ in range(NBUF)],
        [pltpu.VMEM((CHUNK, 1, D_MODEL), jnp.float32) for _ in range(NBUF)],
        [pltpu.SemaphoreType.DMA for _ in range(NBUF)],
        [pltpu.SemaphoreType.DMA for _ in range(NBUF)],
        [pltpu.SemaphoreType.DMA for _ in range(NBUF)],
    ],
    compiler_params=pltpu.CompilerParams(use_tc_tiling_on_sc=False),
)
def _pe_add(x_hbm, ts_hbm, pe_hbm, out_hbm, idx_v, xbufs, pebufs,
            semx, semg, sems):
    wid = lax.axis_index("s") * NC + lax.axis_index("c")
    lbase = 1 + wid * (ROWS_PER_TILE // B)

    pltpu.sync_copy(ts_hbm.at[wid], idx_v)
    for c in range(NCHUNK):
        v = idx_v[c, :]
        idx_v[c, :] = jnp.minimum(jnp.maximum(v, 0), MAX_LEN - 1)

    @pl.when(wid == 0)
    def _():
        pltpu.sync_copy(x_hbm.at[pl.ds(0, 1)], xbufs[0].at[pl.ds(0, 1)])
        pltpu.sync_copy(xbufs[0].at[pl.ds(0, 1)], out_hbm.at[pl.ds(0, 1)])

    def l0(c):
        return lbase + c * LCHUNK

    def issue(c):
        b = c % NBUF
        cx = pltpu.async_copy(x_hbm.at[pl.ds(l0(c), LCHUNK)], xbufs[b],
                              semx[b])
        cg = pltpu.async_copy(pe_hbm.at[idx_v.at[c]], pebufs[b], semg[b])
        return cx, cg

    cp_x = [None] * NCHUNK
    cp_g = [None] * NCHUNK
    cp_s = [None] * NCHUNK
    for c in range(min(PRE, NCHUNK)):
        cp_x[c], cp_g[c] = issue(c)
    for c in range(NCHUNK):
        b = c % NBUF
        cp_x[c].wait()
        cp_g[c].wait()
        xb, pb = xbufs[b], pebufs[b]

        def row_body(r, carry, xb=xb, pb=pb):
            li = lax.shift_right_logical(r, 2)
            bi = lax.bitwise_and(r, 3)
            for j in range(VECS_PER_ROW):
                sl = pl.ds(j * LANES, LANES)
                xb[li, bi, sl] = xb[li, bi, sl] + pb[r, 0, sl]
            return carry

        lax.fori_loop(0, CHUNK, row_body, 0)
        cp_s[c] = pltpu.async_copy(xb, out_hbm.at[pl.ds(l0(c), LCHUNK)],
                                   sems[b])
        nxt = c + PRE
        if nxt < NCHUNK:
            if nxt - NBUF >= 0:
                cp_s[nxt - NBUF].wait()
            cp_x[nxt], cp_g[nxt] = issue(nxt)
    for c in range(max(0, NCHUNK - NBUF), NCHUNK):
        cp_s[c].wait()


def kernel(x, timestamps, pe):
    ts3 = timestamps.T.reshape(NW, NCHUNK, CHUNK)
    return _pe_add(x, ts3, pe)
